# Initial kernel scaffold; baseline (speedup 1.0000x reference)
import jax
import jax.numpy as jnp
from jax.experimental import pallas as pl


def kernel(x, edge_index, edge_attr, u, batch, mlp1_W0, mlp1_b0, mlp1_W1, mlp1_b1, mlp1_W2, mlp1_b2, mlp1_W3, mlp1_b3, mlp1_W4, mlp1_b4, mlp2_W0, mlp2_b0, mlp2_W1, mlp2_b1, mlp2_W2, mlp2_b2, mlp2_W3, mlp2_b3, mlp2_W4, mlp2_b4):
    raise NotImplementedError("write your pallas kernel here")


# speedup vs baseline: 1.3901x; 1.3901x over previous
import functools

import jax
import jax.numpy as jnp
from jax import lax
from jax.experimental import pallas as pl
from jax.experimental.pallas import tpu as pltpu
from jax.experimental.pallas import tpu_sc as plsc

N = 10000
E = 160000
NODE_IN = 256
LEAK = 0.1

NUM_CORES = 2
NUM_SUBCORES = 16
NUM_WORKERS = NUM_CORES * NUM_SUBCORES
EDGES_PER_WORKER = E // NUM_WORKERS
CHUNK = 1000
PAD_W = 16


def _dot(a, b):
    return lax.dot_general(a, b, (((1,), (0,)), ((), ())),
                           preferred_element_type=jnp.float32,
                           precision=lax.Precision.HIGHEST)


def _leaky(h):
    return jnp.where(h >= 0, h, LEAK * h)


def _node_mm_body(x_ref, w1x_ref, b10_ref, w2x_ref, xw1_ref, xw2_ref):
    xb = x_ref[...]
    xw1_ref[...] = _dot(xb, w1x_ref[...]) + b10_ref[...]
    xw2_ref[...] = _dot(xb, w2x_ref[...])


def _node_matmuls(x, w1x, b10, w2x):
    blk = 1000
    grid = (N // blk,)
    full = lambda shape: pl.BlockSpec(shape, lambda i: (0,) * len(shape))
    return pl.pallas_call(
        _node_mm_body,
        grid=grid,
        in_specs=[
            pl.BlockSpec((blk, NODE_IN), lambda i: (i, 0)),
            full((NODE_IN, 64)),
            full((1, 64)),
            full((NODE_IN, 64)),
        ],
        out_specs=[
            pl.BlockSpec((blk, 64), lambda i: (i, 0)),
            pl.BlockSpec((blk, 64), lambda i: (i, 0)),
        ],
        out_shape=[
            jax.ShapeDtypeStruct((N, 64), jnp.float32),
            jax.ShapeDtypeStruct((N, 64), jnp.float32),
        ],
    )(x, w1x, b10.reshape(1, 64), w2x)


def _sc_gather(table, idx):
    mesh = plsc.VectorSubcoreMesh(core_axis_name="c", subcore_axis_name="s")

    @functools.partial(
        pl.kernel,
        out_type=jax.ShapeDtypeStruct((E, 64), jnp.float32),
        mesh=mesh,
        scratch_types=[
            pltpu.VMEM((CHUNK,), jnp.int32),
            pltpu.VMEM((CHUNK, 64), jnp.float32),
            pltpu.SemaphoreType.DMA,
        ],
        compiler_params=pltpu.CompilerParams(use_tc_tiling_on_sc=False),
    )
    def k(table_hbm, idx_hbm, out_hbm, idx_v, rows_v, sem):
        wid = lax.axis_index("s") * NUM_CORES + lax.axis_index("c")
        base = wid * EDGES_PER_WORKER

        def body(j, carry):
            off = base + j * CHUNK
            pltpu.sync_copy(idx_hbm.at[pl.ds(off, CHUNK)], idx_v)
            pltpu.async_copy(table_hbm.at[idx_v], rows_v, sem).wait()
            pltpu.sync_copy(rows_v, out_hbm.at[pl.ds(off, CHUNK)])
            return carry

        lax.fori_loop(0, EDGES_PER_WORKER // CHUNK, body, 0)

    return k(table, idx)


def _edge_mlp_body(g_ref, ea_ref, w1e_ref, w11_ref, b11_ref, w12_ref, b12_ref,
                   w13_ref, b13_ref, w14p_ref, b14p_ref, out_ref):
    h = _leaky(g_ref[...] + _dot(ea_ref[...], w1e_ref[...]))
    h = _leaky(_dot(h, w11_ref[...]) + b11_ref[...])
    h = _leaky(_dot(h, w12_ref[...]) + b12_ref[...])
    h = _leaky(_dot(h, w13_ref[...]) + b13_ref[...])
    out_ref[...] = _dot(h, w14p_ref[...]) + b14p_ref[...]


def _edge_mlp(g, ea, w1e, w11, b11, w12, b12, w13, b13, w14p, b14p):
    blk = 2000
    grid = (E // blk,)
    full = lambda shape: pl.BlockSpec(shape, lambda i: (0,) * len(shape))
    return pl.pallas_call(
        _edge_mlp_body,
        grid=grid,
        in_specs=[
            pl.BlockSpec((blk, 64), lambda i: (i, 0)),
            pl.BlockSpec((blk, 16), lambda i: (i, 0)),
            full((16, 64)),
            full((64, 32)), full((1, 32)),
            full((32, 16)), full((1, 16)),
            full((16, 8)), full((1, 8)),
            full((8, PAD_W)), full((1, PAD_W)),
        ],
        out_specs=pl.BlockSpec((blk, PAD_W), lambda i: (i, 0)),
        out_shape=jax.ShapeDtypeStruct((E, PAD_W), jnp.float32),
    )(g, ea, w1e, w11, b11.reshape(1, -1), w12, b12.reshape(1, -1),
      w13, b13.reshape(1, -1), w14p, b14p)


def _sc_scatter(vals, col, zeros):
    mesh = plsc.VectorSubcoreMesh(core_axis_name="c", subcore_axis_name="s")

    @functools.partial(
        pl.kernel,
        out_type=jax.ShapeDtypeStruct((NUM_CORES, N, PAD_W), jnp.float32),
        mesh=mesh,
        scratch_types=[
            pltpu.VMEM((CHUNK,), jnp.int32),
            pltpu.VMEM((CHUNK, PAD_W), jnp.float32),
            pltpu.VMEM_SHARED((N, PAD_W), jnp.float32),
        ],
        compiler_params=pltpu.CompilerParams(use_tc_tiling_on_sc=False),
    )
    def k(vals_hbm, col_hbm, zeros_hbm, out_hbm, idx_v, vals_v, accum_sh):
        c = lax.axis_index("c")
        s = lax.axis_index("s")

        @pl.when(s == 0)
        def _():
            pltpu.sync_copy(zeros_hbm, accum_sh)

        plsc.subcore_barrier()

        wid = s * NUM_CORES + c
        base = wid * EDGES_PER_WORKER

        def body(j, carry):
            off = base + j * CHUNK
            pltpu.sync_copy(col_hbm.at[pl.ds(off, CHUNK)], idx_v)
            pltpu.sync_copy(vals_hbm.at[pl.ds(off, CHUNK)], vals_v)
            pltpu.sync_copy(vals_v, accum_sh.at[idx_v], add=True)
            return carry

        lax.fori_loop(0, EDGES_PER_WORKER // CHUNK, body, 0)

        plsc.subcore_barrier()

        @pl.when(s == 0)
        def _():
            pltpu.sync_copy(accum_sh, out_hbm.at[c])

    return k(vals, col, zeros)


def _node2_body(xw2_ref, p0_ref, p1_ref, w2m_ref, b20_ref, w21_ref, b21_ref,
                w22_ref, b22_ref, w23_ref, b23_ref, w24_ref, b24_ref, out_ref):
    acc = p0_ref[...] + p1_ref[...]
    sums = acc[:, 0:2]
    cnt = acc[:, 2:3]
    mean = sums / jnp.maximum(cnt, 1.0)
    h = _leaky(xw2_ref[...] + _dot(mean, w2m_ref[...]) + b20_ref[...])
    h = _leaky(_dot(h, w21_ref[...]) + b21_ref[...])
    h = _leaky(_dot(h, w22_ref[...]) + b22_ref[...])
    h = _leaky(_dot(h, w23_ref[...]) + b23_ref[...])
    out_ref[...] = _dot(h, w24_ref[...]) + b24_ref[...]


def _node_mlp2(xw2, p0, p1, w2m, b20, w21, b21, w22, b22, w23, b23, w24, b24):
    blk = 1000
    grid = (N // blk,)
    full = lambda shape: pl.BlockSpec(shape, lambda i: (0,) * len(shape))
    return pl.pallas_call(
        _node2_body,
        grid=grid,
        in_specs=[
            pl.BlockSpec((blk, 64), lambda i: (i, 0)),
            pl.BlockSpec((blk, PAD_W), lambda i: (i, 0)),
            pl.BlockSpec((blk, PAD_W), lambda i: (i, 0)),
            full((2, 64)), full((1, 64)),
            full((64, 32)), full((1, 32)),
            full((32, 16)), full((1, 16)),
            full((16, 8)), full((1, 8)),
            full((8, 2)), full((1, 2)),
        ],
        out_specs=pl.BlockSpec((blk, 2), lambda i: (i, 0)),
        out_shape=jax.ShapeDtypeStruct((N, 2), jnp.float32),
    )(xw2, p0, p1, w2m, b20.reshape(1, -1), w21, b21.reshape(1, -1),
      w22, b22.reshape(1, -1), w23, b23.reshape(1, -1), w24, b24.reshape(1, -1))


def kernel(x, edge_index, edge_attr, u, batch,
           mlp1_W0, mlp1_b0, mlp1_W1, mlp1_b1, mlp1_W2, mlp1_b2,
           mlp1_W3, mlp1_b3, mlp1_W4, mlp1_b4,
           mlp2_W0, mlp2_b0, mlp2_W1, mlp2_b1, mlp2_W2, mlp2_b2,
           mlp2_W3, mlp2_b3, mlp2_W4, mlp2_b4):
    row = edge_index[0]
    col = edge_index[1]

    w1x = mlp1_W0[:NODE_IN]
    w1e = mlp1_W0[NODE_IN:]
    w2x = mlp2_W0[:NODE_IN]
    w2m = mlp2_W0[NODE_IN:]

    w14p = jnp.zeros((8, PAD_W), jnp.float32).at[:, 0:2].set(mlp1_W4)
    b14p = jnp.zeros((1, PAD_W), jnp.float32).at[0, 0:2].set(mlp1_b4).at[0, 2].set(1.0)

    xw1, xw2 = _node_matmuls(x, w1x, mlp1_b0, w2x)
    g = _sc_gather(xw1, row)
    vals = _edge_mlp(g, edge_attr, w1e, mlp1_W1, mlp1_b1, mlp1_W2, mlp1_b2,
                     mlp1_W3, mlp1_b3, w14p, b14p)
    zeros = jnp.zeros((N, PAD_W), jnp.float32)
    acc = _sc_scatter(vals, col, zeros)
    out = _node_mlp2(xw2, acc[0], acc[1], w2m, mlp2_b0, mlp2_W1, mlp2_b1,
                     mlp2_W2, mlp2_b2, mlp2_W3, mlp2_b3, mlp2_W4, mlp2_b4)
    return out


# speedup vs baseline: 3.1364x; 2.2563x over previous
import functools

import jax
import jax.numpy as jnp
from jax import lax
from jax.experimental import pallas as pl
from jax.experimental.pallas import tpu as pltpu
from jax.experimental.pallas import tpu_sc as plsc

N = 10000
E = 160000
NODE_IN = 256
LEAK = 0.1

NUM_CORES = 2
NUM_SUBCORES = 16
NUM_WORKERS = NUM_CORES * NUM_SUBCORES
EDGES_PER_WORKER = E // NUM_WORKERS
CHUNK = 1000
PAD_W = 16


def _dot(a, b):
    return lax.dot_general(a, b, (((1,), (0,)), ((), ())),
                           preferred_element_type=jnp.float32,
                           precision=lax.Precision.DEFAULT)


def _leaky(h):
    return jnp.where(h >= 0, h, LEAK * h)


def _node_mm_body(x_ref, w1x_ref, b10_ref, w2x_ref, xw1_ref, xw2_ref):
    xb = x_ref[...]
    xw1_ref[...] = _dot(xb, w1x_ref[...]) + b10_ref[...]
    xw2_ref[...] = _dot(xb, w2x_ref[...])


def _node_matmuls(x, w1x, b10, w2x):
    blk = 1000
    grid = (N // blk,)
    full = lambda shape: pl.BlockSpec(shape, lambda i: (0,) * len(shape))
    return pl.pallas_call(
        _node_mm_body,
        grid=grid,
        in_specs=[
            pl.BlockSpec((blk, NODE_IN), lambda i: (i, 0)),
            full((NODE_IN, 64)),
            full((1, 64)),
            full((NODE_IN, 64)),
        ],
        out_specs=[
            pl.BlockSpec((blk, 64), lambda i: (i, 0)),
            pl.BlockSpec((blk, 64), lambda i: (i, 0)),
        ],
        out_shape=[
            jax.ShapeDtypeStruct((N, 64), jnp.float32),
            jax.ShapeDtypeStruct((N, 64), jnp.float32),
        ],
    )(x, w1x, b10.reshape(1, 64), w2x)


def _sc_gather(table, idx):
    mesh = plsc.VectorSubcoreMesh(core_axis_name="c", subcore_axis_name="s")

    @functools.partial(
        pl.kernel,
        out_type=jax.ShapeDtypeStruct((E, 64), jnp.float32),
        mesh=mesh,
        scratch_types=[
            pltpu.VMEM((CHUNK,), jnp.int32),
            pltpu.VMEM((CHUNK, 64), jnp.float32),
            pltpu.SemaphoreType.DMA,
        ],
        compiler_params=pltpu.CompilerParams(use_tc_tiling_on_sc=False),
    )
    def k(table_hbm, idx_hbm, out_hbm, idx_v, rows_v, sem):
        wid = lax.axis_index("s") * NUM_CORES + lax.axis_index("c")
        base = wid * EDGES_PER_WORKER

        def body(j, carry):
            off = base + j * CHUNK
            pltpu.sync_copy(idx_hbm.at[pl.ds(off, CHUNK)], idx_v)
            pltpu.async_copy(table_hbm.at[idx_v], rows_v, sem).wait()
            pltpu.sync_copy(rows_v, out_hbm.at[pl.ds(off, CHUNK)])
            return carry

        lax.fori_loop(0, EDGES_PER_WORKER // CHUNK, body, 0)

    return k(table, idx)


def _edge_mlp_body(g_ref, ea_ref, w1e_ref, w11_ref, b11_ref, w12_ref, b12_ref,
                   w13_ref, b13_ref, w14p_ref, b14p_ref, out_ref):
    h = _leaky(g_ref[...] + _dot(ea_ref[...], w1e_ref[...]))
    h = _leaky(_dot(h, w11_ref[...]) + b11_ref[...])
    h = _leaky(_dot(h, w12_ref[...]) + b12_ref[...])
    h = _leaky(_dot(h, w13_ref[...]) + b13_ref[...])
    out_ref[...] = _dot(h, w14p_ref[...]) + b14p_ref[...]


def _edge_mlp(g, ea, w1e, w11, b11, w12, b12, w13, b13, w14p, b14p):
    blk = 2000
    grid = (E // blk,)
    full = lambda shape: pl.BlockSpec(shape, lambda i: (0,) * len(shape))
    return pl.pallas_call(
        _edge_mlp_body,
        grid=grid,
        in_specs=[
            pl.BlockSpec((blk, 64), lambda i: (i, 0)),
            pl.BlockSpec((blk, 16), lambda i: (i, 0)),
            full((16, 64)),
            full((64, 32)), full((1, 32)),
            full((32, 16)), full((1, 16)),
            full((16, 8)), full((1, 8)),
            full((8, PAD_W)), full((1, PAD_W)),
        ],
        out_specs=pl.BlockSpec((blk, PAD_W), lambda i: (i, 0)),
        out_shape=jax.ShapeDtypeStruct((E, PAD_W), jnp.float32),
    )(g, ea, w1e, w11, b11.reshape(1, -1), w12, b12.reshape(1, -1),
      w13, b13.reshape(1, -1), w14p, b14p)


def _sc_scatter(vals, col, zeros):
    mesh = plsc.VectorSubcoreMesh(core_axis_name="c", subcore_axis_name="s")

    @functools.partial(
        pl.kernel,
        out_type=jax.ShapeDtypeStruct((NUM_CORES, N, PAD_W), jnp.float32),
        mesh=mesh,
        scratch_types=[
            pltpu.VMEM((CHUNK,), jnp.int32),
            pltpu.VMEM((CHUNK, PAD_W), jnp.float32),
            pltpu.VMEM_SHARED((N, PAD_W), jnp.float32),
        ],
        compiler_params=pltpu.CompilerParams(use_tc_tiling_on_sc=False),
    )
    def k(vals_hbm, col_hbm, zeros_hbm, out_hbm, idx_v, vals_v, accum_sh):
        c = lax.axis_index("c")
        s = lax.axis_index("s")

        @pl.when(s == 0)
        def _():
            pltpu.sync_copy(zeros_hbm, accum_sh)

        plsc.subcore_barrier()

        wid = s * NUM_CORES + c
        base = wid * EDGES_PER_WORKER

        def body(j, carry):
            off = base + j * CHUNK
            pltpu.sync_copy(col_hbm.at[pl.ds(off, CHUNK)], idx_v)
            pltpu.sync_copy(vals_hbm.at[pl.ds(off, CHUNK)], vals_v)
            pltpu.sync_copy(vals_v, accum_sh.at[idx_v], add=True)
            return carry

        lax.fori_loop(0, EDGES_PER_WORKER // CHUNK, body, 0)

        plsc.subcore_barrier()

        @pl.when(s == 0)
        def _():
            pltpu.sync_copy(accum_sh, out_hbm.at[c])

    return k(vals, col, zeros)


def _node2_body(xw2_ref, p0_ref, p1_ref, w2m_ref, b20_ref, w21_ref, b21_ref,
                w22_ref, b22_ref, w23_ref, b23_ref, w24_ref, b24_ref, out_ref):
    acc = p0_ref[...] + p1_ref[...]
    sums = acc[:, 0:2]
    cnt = acc[:, 2:3]
    mean = sums / jnp.maximum(cnt, 1.0)
    h = _leaky(xw2_ref[...] + _dot(mean, w2m_ref[...]) + b20_ref[...])
    h = _leaky(_dot(h, w21_ref[...]) + b21_ref[...])
    h = _leaky(_dot(h, w22_ref[...]) + b22_ref[...])
    h = _leaky(_dot(h, w23_ref[...]) + b23_ref[...])
    out_ref[...] = _dot(h, w24_ref[...]) + b24_ref[...]


def _node_mlp2(xw2, p0, p1, w2m, b20, w21, b21, w22, b22, w23, b23, w24, b24):
    blk = 1000
    grid = (N // blk,)
    full = lambda shape: pl.BlockSpec(shape, lambda i: (0,) * len(shape))
    return pl.pallas_call(
        _node2_body,
        grid=grid,
        in_specs=[
            pl.BlockSpec((blk, 64), lambda i: (i, 0)),
            pl.BlockSpec((blk, PAD_W), lambda i: (i, 0)),
            pl.BlockSpec((blk, PAD_W), lambda i: (i, 0)),
            full((2, 64)), full((1, 64)),
            full((64, 32)), full((1, 32)),
            full((32, 16)), full((1, 16)),
            full((16, 8)), full((1, 8)),
            full((8, 2)), full((1, 2)),
        ],
        out_specs=pl.BlockSpec((blk, 2), lambda i: (i, 0)),
        out_shape=jax.ShapeDtypeStruct((N, 2), jnp.float32),
    )(xw2, p0, p1, w2m, b20.reshape(1, -1), w21, b21.reshape(1, -1),
      w22, b22.reshape(1, -1), w23, b23.reshape(1, -1), w24, b24.reshape(1, -1))


def kernel(x, edge_index, edge_attr, u, batch,
           mlp1_W0, mlp1_b0, mlp1_W1, mlp1_b1, mlp1_W2, mlp1_b2,
           mlp1_W3, mlp1_b3, mlp1_W4, mlp1_b4,
           mlp2_W0, mlp2_b0, mlp2_W1, mlp2_b1, mlp2_W2, mlp2_b2,
           mlp2_W3, mlp2_b3, mlp2_W4, mlp2_b4):
    row = edge_index[0]
    col = edge_index[1]

    w1x = mlp1_W0[:NODE_IN]
    w1e = mlp1_W0[NODE_IN:]
    w2x = mlp2_W0[:NODE_IN]
    w2m = mlp2_W0[NODE_IN:]

    w14p = jnp.zeros((8, PAD_W), jnp.float32).at[:, 0:2].set(mlp1_W4)
    b14p = jnp.zeros((1, PAD_W), jnp.float32).at[0, 0:2].set(mlp1_b4).at[0, 2].set(1.0)

    xw1, xw2 = _node_matmuls(x, w1x, mlp1_b0, w2x)
    g = _sc_gather(xw1, row)
    vals = _edge_mlp(g, edge_attr, w1e, mlp1_W1, mlp1_b1, mlp1_W2, mlp1_b2,
                     mlp1_W3, mlp1_b3, w14p, b14p)
    zeros = jnp.zeros((N, PAD_W), jnp.float32)
    acc = _sc_scatter(vals, col, zeros)
    out = _node_mlp2(xw2, acc[0], acc[1], w2m, mlp2_b0, mlp2_W1, mlp2_b1,
                     mlp2_W2, mlp2_b2, mlp2_W3, mlp2_b3, mlp2_W4, mlp2_b4)
    return out
